# baseline (speedup 1.0000x reference)
import jax
import jax.numpy as jnp
from jax.experimental import pallas as pl
from jax.experimental.pallas import tpu as pltpu

S = 8192
DIM = 2048
N_EXPERT = 64
K = 2
BLK = 512
GRID = S // BLK
NBUF = 4


def _outer(x_hbm, w_ref, b_ref, ids_ref, vals_ref, aux_ref,
           psum_ref, cnt_ref):
    psum_ref[...] = jnp.zeros_like(psum_ref)
    cnt_ref[...] = jnp.zeros_like(cnt_ref)

    def body(x_blk):
        i = pl.program_id(0)
        logits = jnp.dot(x_blk[...], w_ref[...],
                         preferred_element_type=jnp.float32) + b_ref[...]

        m = jnp.max(logits, axis=1, keepdims=True)
        e = jnp.exp(logits - m)
        s = jnp.sum(e, axis=1, keepdims=True)
        prob = e / s

        lane = jax.lax.broadcasted_iota(jnp.int32, prob.shape, 1)
        i1 = jnp.argmax(prob, axis=1).astype(jnp.int32)
        v1 = jnp.max(prob, axis=1)
        masked = jnp.where(lane == i1[:, None], -1.0, prob)
        i2 = jnp.argmax(masked, axis=1).astype(jnp.int32)
        v2 = jnp.max(masked, axis=1)

        ids_ref[pl.ds(i * BLK, BLK), :] = jnp.stack([i1, i2], axis=1)
        vals_ref[pl.ds(i * BLK, BLK), :] = jnp.stack([v1, v2], axis=1)

        one_hot = (lane == i1[:, None]).astype(jnp.float32)
        psum_ref[...] += jnp.sum(prob, axis=0, keepdims=True)
        cnt_ref[...] += jnp.sum(one_hot, axis=0, keepdims=True)

    pipe = pltpu.emit_pipeline(
        body,
        grid=(GRID,),
        in_specs=[pl.BlockSpec((BLK, DIM), lambda i: (i, 0),
                               pipeline_mode=pl.Buffered(buffer_count=NBUF))],
    )
    pipe(x_hbm)

    aux_ref[...] = (float(N_EXPERT) / (S * S)) * jnp.sum(
        psum_ref[...] * cnt_ref[...], axis=1, keepdims=True)


@jax.jit
def kernel(x, W, b):
    ids, vals, aux = pl.pallas_call(
        _outer,
        in_specs=[
            pl.BlockSpec(memory_space=pl.ANY),
            pl.BlockSpec(memory_space=pltpu.MemorySpace.VMEM),
            pl.BlockSpec(memory_space=pltpu.MemorySpace.VMEM),
        ],
        out_specs=[
            pl.BlockSpec(memory_space=pltpu.MemorySpace.VMEM),
            pl.BlockSpec(memory_space=pltpu.MemorySpace.VMEM),
            pl.BlockSpec(memory_space=pltpu.MemorySpace.VMEM),
        ],
        out_shape=[
            jax.ShapeDtypeStruct((S, K), jnp.int32),
            jax.ShapeDtypeStruct((S, K), jnp.float32),
            jax.ShapeDtypeStruct((1, 1), jnp.float32),
        ],
        scratch_shapes=[
            pltpu.VMEM((1, N_EXPERT), jnp.float32),
            pltpu.VMEM((1, N_EXPERT), jnp.float32),
        ],
    )(x, W, b.reshape(1, N_EXPERT))
    return ids, vals, aux[0, 0]
